# baseline (speedup 1.0000x reference)
import functools

import jax
import jax.numpy as jnp
from jax import lax
from jax.experimental import pallas as pl
from jax.experimental.pallas import tpu as pltpu
from jax.experimental.pallas import tpu_sc as plsc

N_VARS = 10000
N_FACTORS = 10000
N_EDGES = 320000
D = 128

NC = 2
NS = 16
NW = NC * NS
EPW = N_EDGES // NW
C = 40
NCHUNK = EPW // C
SB = 25
NPLANE = 3
NSET = 4
SUB_ROWS = 624
LAST_ROWS = N_VARS - (NS - 1) * SUB_ROWS
LAST_OFF = (NS - 1) * SUB_ROWS

ROW_BLK = 1000
GRID = N_VARS // ROW_BLK


def _pre_body(v_ref, f_ref, w1_ref, w2_ref, bm_ref, t_ref):
    i = pl.program_id(0)

    @pl.when(i < GRID)
    def _():
        t_ref[...] = jnp.dot(v_ref[...], w1_ref[...],
                             preferred_element_type=jnp.float32) + bm_ref[...]

    @pl.when(i >= GRID)
    def _():
        t_ref[...] = jnp.dot(f_ref[...], w2_ref[...],
                             preferred_element_type=jnp.float32)


def _pre(variables, factors, w1, w2, b_msg):
    return pl.pallas_call(
        _pre_body,
        grid=(2 * GRID,),
        in_specs=[
            pl.BlockSpec((ROW_BLK, D), lambda i: (i % GRID, 0)),
            pl.BlockSpec((ROW_BLK, D), lambda i: (i % GRID, 0)),
            pl.BlockSpec((D, D), lambda i: (0, 0)),
            pl.BlockSpec((D, D), lambda i: (0, 0)),
            pl.BlockSpec((1, D), lambda i: (0, 0)),
        ],
        out_specs=pl.BlockSpec((ROW_BLK, D), lambda i: (i, 0)),
        out_shape=jax.ShapeDtypeStruct((N_VARS + N_FACTORS, D), jnp.float32),
    )(variables, factors, w1, w2, b_msg)


def _sc_body(t_hbm, comb_hbm, snd_hbm, zeros_hbm, out_hbm,
             comb_v, snd_v, g0, g1, g2, g3, acc,
             sg0, sg1, sg2, sg3, ss0, ss1, ss2, ss3):
    c = lax.axis_index("c")
    s = lax.axis_index("s")
    wid = c * NS + s

    @pl.when(s < NS - 1)
    def _():
        pltpu.sync_copy(zeros_hbm.at[pl.ds(0, SUB_ROWS)],
                        acc.at[pl.ds(s * SUB_ROWS, SUB_ROWS)])

    @pl.when(s == NS - 1)
    def _():
        pltpu.sync_copy(zeros_hbm, acc.at[pl.ds(LAST_OFF, LAST_ROWS)])

    pltpu.sync_copy(comb_hbm.at[wid, pl.ds(0, SB)], comb_v.at[0])
    pltpu.sync_copy(comb_hbm.at[wid, pl.ds(SB, SB)], comb_v.at[1])
    pltpu.sync_copy(snd_hbm.at[wid, pl.ds(0, SB)], snd_v.at[0])
    pltpu.sync_copy(snd_hbm.at[wid, pl.ds(SB, SB)], snd_v.at[1])
    plsc.subcore_barrier()

    gbufs = (g0, g1, g2, g3)
    gsems, ssems = (sg0, sg1, sg2, sg3), (ss0, ss1, ss2, ss3)

    def cidx(x):
        return comb_v.at[(x // SB) % NPLANE, x % SB]

    def sidx(x):
        return snd_v.at[(x // SB) % NPLANE, x % SB]

    def issue_gathers(j, k):
        pltpu.async_copy(t_hbm.at[cidx(j)], gbufs[k], gsems[k])

    def wait_gathers(j, k):
        pltpu.make_async_copy(t_hbm.at[cidx(j)], gbufs[k], gsems[k]).wait()

    def compute(k):
        gb = gbufs[k]

        def row(r, carry2):
            for l in range(D // 16):
                sl = pl.ds(l * 16, 16)
                gb[r, sl] = jnp.maximum(gb[r, sl] + gb[r + C, sl], 0.0)
            return carry2

        lax.fori_loop(0, C, row, 0)

    def issue_scatter(j, k):
        pltpu.async_copy(gbufs[k].at[pl.ds(0, C)], acc.at[sidx(j)],
                         ssems[k], add=True)

    def wait_scatter(j, k):
        pltpu.make_async_copy(gbufs[k].at[pl.ds(0, C)], acc.at[sidx(j)],
                              ssems[k]).wait()

    issue_gathers(0, 0)
    issue_gathers(1, 1)
    issue_gathers(2, 2)

    def quad(i, carry):
        for t in range(NSET):
            j = NSET * i + t

            @pl.when(j < NCHUNK)
            def _():
                wait_gathers(j, t)
                compute(t)
                issue_scatter(j, t)

                @pl.when(j >= 1)
                def _():
                    wait_scatter(j - 1, (t + 3) % NSET)

                @pl.when(jnp.logical_and((j + 3) % SB == 0,
                                         j + 3 < NCHUNK))
                def _():
                    x = j + 3
                    p = (x // SB) % NPLANE
                    pltpu.sync_copy(comb_hbm.at[wid, pl.ds(x, SB)],
                                    comb_v.at[p])
                    pltpu.sync_copy(snd_hbm.at[wid, pl.ds(x, SB)],
                                    snd_v.at[p])

                @pl.when(j + 3 < NCHUNK)
                def _():
                    issue_gathers(j + 3, (t + 3) % NSET)

        return carry

    lax.fori_loop(0, (NCHUNK + NSET - 1) // NSET, quad, 0)
    wait_scatter(NCHUNK - 1, (NCHUNK - 1) % NSET)
    plsc.subcore_barrier()

    @pl.when(s < NS - 1)
    def _():
        pltpu.sync_copy(acc.at[pl.ds(s * SUB_ROWS, SUB_ROWS)],
                        out_hbm.at[pl.ds(c * N_VARS + s * SUB_ROWS,
                                         SUB_ROWS)])

    @pl.when(s == NS - 1)
    def _():
        pltpu.sync_copy(acc.at[pl.ds(LAST_OFF, LAST_ROWS)],
                        out_hbm.at[pl.ds(c * N_VARS + LAST_OFF, LAST_ROWS)])


def _sc_edges(t, comb_idx, senders, zeros_rows):
    mesh = plsc.VectorSubcoreMesh(core_axis_name="c", subcore_axis_name="s")
    f = pl.kernel(
        _sc_body,
        out_type=jax.ShapeDtypeStruct((NC * N_VARS, D), jnp.float32),
        mesh=mesh,
        compiler_params=pltpu.CompilerParams(use_tc_tiling_on_sc=False),
        scratch_types=(
            [pltpu.VMEM((NPLANE, SB, 2 * C), jnp.int32)]
            + [pltpu.VMEM((NPLANE, SB, C), jnp.int32)]
            + [pltpu.VMEM((2 * C, D), jnp.float32)] * NSET
            + [pltpu.VMEM_SHARED((N_VARS, D), jnp.float32)]
            + [pltpu.SemaphoreType.DMA] * (2 * NSET)
        ),
    )
    return f(t, comb_idx, senders.reshape(NW, NCHUNK, C), zeros_rows)


def _comb_body(v_ref, p0_ref, p1_ref, wc1_ref, wc2_ref, bc_ref, o_ref):
    v = v_ref[...]
    aggr = p0_ref[...] + p1_ref[...]
    h = (jnp.dot(v, wc1_ref[...], preferred_element_type=jnp.float32)
         + jnp.dot(aggr, wc2_ref[...], preferred_element_type=jnp.float32)
         + bc_ref[...])
    o_ref[...] = v + jnp.maximum(h, 0.0)


def _combine(variables, partials, wc1, wc2, b_comb):
    return pl.pallas_call(
        _comb_body,
        grid=(GRID,),
        in_specs=[
            pl.BlockSpec((ROW_BLK, D), lambda i: (i, 0)),
            pl.BlockSpec((ROW_BLK, D), lambda i: (i, 0)),
            pl.BlockSpec((ROW_BLK, D), lambda i: (i + GRID, 0)),
            pl.BlockSpec((D, D), lambda i: (0, 0)),
            pl.BlockSpec((D, D), lambda i: (0, 0)),
            pl.BlockSpec((1, D), lambda i: (0, 0)),
        ],
        out_specs=pl.BlockSpec((ROW_BLK, D), lambda i: (i, 0)),
        out_shape=jax.ShapeDtypeStruct((N_VARS, D), jnp.float32),
    )(variables, partials, partials, wc1, wc2, b_comb)


def kernel(variables, factors, senders, receivers, edge_attr,
           W_msg, b_msg, W_comb, b_comb):
    del edge_attr
    w1 = W_msg[:D, :]
    w2 = W_msg[D:2 * D, :]
    t = _pre(variables, factors, w1, w2, b_msg.reshape(1, D))
    zeros_rows = jnp.zeros((LAST_ROWS, D), jnp.float32)
    snd3 = senders.astype(jnp.int32).reshape(NW, NCHUNK, C)
    rcv3 = receivers.astype(jnp.int32).reshape(NW, NCHUNK, C)
    comb_idx = jnp.concatenate([snd3, rcv3 + N_VARS], axis=-1)
    partials = _sc_edges(t, comb_idx, senders.astype(jnp.int32), zeros_rows)
    return _combine(variables, partials, W_comb[:D, :], W_comb[D:, :],
                    b_comb.reshape(1, D))


# speedup vs baseline: 1.1782x; 1.1782x over previous
import functools

import jax
import jax.numpy as jnp
from jax import lax
from jax.experimental import pallas as pl
from jax.experimental.pallas import tpu as pltpu
from jax.experimental.pallas import tpu_sc as plsc

N_VARS = 10000
N_FACTORS = 10000
N_EDGES = 320000
D = 128

NC = 2
NS = 16
NW = NC * NS
EPW = N_EDGES // NW
C = 40
NCHUNK = EPW // C
SB = 25
NPLANE = 3
NSET = 4
SUB_ROWS = 624
LAST_ROWS = N_VARS - (NS - 1) * SUB_ROWS
LAST_OFF = (NS - 1) * SUB_ROWS

ROW_BLK = 2000
GRID = N_VARS // ROW_BLK


def _pre_body(v_ref, f_ref, w1_ref, w2_ref, bm_ref, a_ref, b_ref):
    a_ref[...] = jnp.dot(v_ref[...], w1_ref[...],
                         preferred_element_type=jnp.float32) + bm_ref[...]
    b_ref[...] = jnp.dot(f_ref[...], w2_ref[...],
                         preferred_element_type=jnp.float32)


def _pre(variables, factors, w1, w2, b_msg):
    return pl.pallas_call(
        _pre_body,
        grid=(GRID,),
        in_specs=[
            pl.BlockSpec((ROW_BLK, D), lambda i: (i, 0)),
            pl.BlockSpec((ROW_BLK, D), lambda i: (i, 0)),
            pl.BlockSpec((D, D), lambda i: (0, 0)),
            pl.BlockSpec((D, D), lambda i: (0, 0)),
            pl.BlockSpec((1, D), lambda i: (0, 0)),
        ],
        out_specs=[
            pl.BlockSpec((ROW_BLK, D), lambda i: (i, 0)),
            pl.BlockSpec((ROW_BLK, D), lambda i: (i, 0)),
        ],
        out_shape=[
            jax.ShapeDtypeStruct((N_VARS, D), jnp.float32),
            jax.ShapeDtypeStruct((N_FACTORS, D), jnp.float32),
        ],
    )(variables, factors, w1, w2, b_msg)


def _sc_body(a_hbm, b_hbm, snd_hbm, rcv_hbm, zeros_hbm, out_hbm,
             snd_v, rcv_v, a0, a1, a2, a3, b0, b1, b2, b3, acc,
             sg0, sg1, sg2, sg3, ss0, ss1, ss2, ss3):
    c = lax.axis_index("c")
    s = lax.axis_index("s")
    wid = c * NS + s

    pltpu.sync_copy(snd_hbm.at[wid, pl.ds(0, SB)], snd_v.at[0])
    pltpu.sync_copy(snd_hbm.at[wid, pl.ds(SB, SB)], snd_v.at[1])
    pltpu.sync_copy(rcv_hbm.at[wid, pl.ds(0, SB)], rcv_v.at[0])
    pltpu.sync_copy(rcv_hbm.at[wid, pl.ds(SB, SB)], rcv_v.at[1])

    abufs, bbufs = (a0, a1, a2, a3), (b0, b1, b2, b3)
    gsems, ssems = (sg0, sg1, sg2, sg3), (ss0, ss1, ss2, ss3)

    def sidx(x):
        return snd_v.at[(x // SB) % NPLANE, x % SB]

    def ridx(x):
        return rcv_v.at[(x // SB) % NPLANE, x % SB]

    HALves = ((0, 24), (24, 16))

    def shalf(x, off, ln):
        return snd_v.at[(x // SB) % NPLANE, x % SB, pl.ds(off, ln)]

    def rhalf(x, off, ln):
        return rcv_v.at[(x // SB) % NPLANE, x % SB, pl.ds(off, ln)]

    def issue_gathers(j, k):
        for off, ln in HALves:
            hs = pl.ds(off, ln)
            pltpu.async_copy(a_hbm.at[shalf(j, off, ln)], abufs[k].at[hs],
                             gsems[k])
            pltpu.async_copy(b_hbm.at[rhalf(j, off, ln)], bbufs[k].at[hs],
                             gsems[k])

    def wait_gathers(j, k):
        for off, ln in HALves:
            hs = pl.ds(off, ln)
            pltpu.make_async_copy(a_hbm.at[shalf(j, off, ln)],
                                  abufs[k].at[hs], gsems[k]).wait()
            pltpu.make_async_copy(b_hbm.at[rhalf(j, off, ln)],
                                  bbufs[k].at[hs], gsems[k]).wait()

    def compute(k):
        ab, bb = abufs[k], bbufs[k]

        def row(r, carry2):
            for l in range(D // 16):
                sl = pl.ds(l * 16, 16)
                bb[r, sl] = jnp.maximum(ab[r, sl] + bb[r, sl], 0.0)
            return carry2

        lax.fori_loop(0, C, row, 0)

    def issue_scatter(j, k):
        pltpu.async_copy(bbufs[k], acc.at[sidx(j)], ssems[k], add=True)

    def wait_scatter(j, k):
        pltpu.make_async_copy(bbufs[k], acc.at[sidx(j)], ssems[k]).wait()

    issue_gathers(0, 0)
    issue_gathers(1, 1)
    issue_gathers(2, 2)

    @pl.when(s < NS - 1)
    def _():
        pltpu.sync_copy(zeros_hbm.at[pl.ds(0, SUB_ROWS)],
                        acc.at[pl.ds(s * SUB_ROWS, SUB_ROWS)])

    @pl.when(s == NS - 1)
    def _():
        pltpu.sync_copy(zeros_hbm, acc.at[pl.ds(LAST_OFF, LAST_ROWS)])

    plsc.subcore_barrier()

    def quad(i, carry):
        for t in range(NSET):
            j = NSET * i + t

            @pl.when(j < NCHUNK)
            def _():
                wait_gathers(j, t)
                compute(t)
                issue_scatter(j, t)

                @pl.when(j >= 1)
                def _():
                    wait_scatter(j - 1, (t + 3) % NSET)

                @pl.when(jnp.logical_and((j + 3) % SB == 0,
                                         j + 3 < NCHUNK))
                def _():
                    x = j + 3
                    p = (x // SB) % NPLANE
                    pltpu.sync_copy(snd_hbm.at[wid, pl.ds(x, SB)],
                                    snd_v.at[p])
                    pltpu.sync_copy(rcv_hbm.at[wid, pl.ds(x, SB)],
                                    rcv_v.at[p])

                @pl.when(j + 3 < NCHUNK)
                def _():
                    issue_gathers(j + 3, (t + 3) % NSET)

        return carry

    lax.fori_loop(0, (NCHUNK + NSET - 1) // NSET, quad, 0)
    wait_scatter(NCHUNK - 1, (NCHUNK - 1) % NSET)
    plsc.subcore_barrier()

    @pl.when(s < NS - 1)
    def _():
        pltpu.sync_copy(acc.at[pl.ds(s * SUB_ROWS, SUB_ROWS)],
                        out_hbm.at[pl.ds(c * N_VARS + s * SUB_ROWS,
                                         SUB_ROWS)])

    @pl.when(s == NS - 1)
    def _():
        pltpu.sync_copy(acc.at[pl.ds(LAST_OFF, LAST_ROWS)],
                        out_hbm.at[pl.ds(c * N_VARS + LAST_OFF, LAST_ROWS)])


def _sc_edges(a, b, senders, receivers, zeros_rows):
    mesh = plsc.VectorSubcoreMesh(core_axis_name="c", subcore_axis_name="s")
    f = pl.kernel(
        _sc_body,
        out_type=jax.ShapeDtypeStruct((NC * N_VARS, D), jnp.float32),
        mesh=mesh,
        compiler_params=pltpu.CompilerParams(use_tc_tiling_on_sc=False),
        scratch_types=(
            [pltpu.VMEM((NPLANE, SB, C), jnp.int32)] * 2
            + [pltpu.VMEM((C, D), jnp.float32)] * (2 * NSET)
            + [pltpu.VMEM_SHARED((N_VARS, D), jnp.float32)]
            + [pltpu.SemaphoreType.DMA] * (2 * NSET)
        ),
    )
    return f(a, b, senders.reshape(NW, NCHUNK, C),
             receivers.reshape(NW, NCHUNK, C), zeros_rows)


def _comb_body(v_ref, p0_ref, p1_ref, wc1_ref, wc2_ref, bc_ref, o_ref):
    v = v_ref[...]
    aggr = p0_ref[...] + p1_ref[...]
    h = (jnp.dot(v, wc1_ref[...], preferred_element_type=jnp.float32)
         + jnp.dot(aggr, wc2_ref[...], preferred_element_type=jnp.float32)
         + bc_ref[...])
    o_ref[...] = v + jnp.maximum(h, 0.0)


def _combine(variables, partials, wc1, wc2, b_comb):
    return pl.pallas_call(
        _comb_body,
        grid=(GRID,),
        in_specs=[
            pl.BlockSpec((ROW_BLK, D), lambda i: (i, 0)),
            pl.BlockSpec((ROW_BLK, D), lambda i: (i, 0)),
            pl.BlockSpec((ROW_BLK, D), lambda i: (i + GRID, 0)),
            pl.BlockSpec((D, D), lambda i: (0, 0)),
            pl.BlockSpec((D, D), lambda i: (0, 0)),
            pl.BlockSpec((1, D), lambda i: (0, 0)),
        ],
        out_specs=pl.BlockSpec((ROW_BLK, D), lambda i: (i, 0)),
        out_shape=jax.ShapeDtypeStruct((N_VARS, D), jnp.float32),
    )(variables, partials, partials, wc1, wc2, b_comb)


def kernel(variables, factors, senders, receivers, edge_attr,
           W_msg, b_msg, W_comb, b_comb):
    del edge_attr
    w1 = W_msg[:D, :]
    w2 = W_msg[D:2 * D, :]
    a, b = _pre(variables, factors, w1, w2, b_msg.reshape(1, D))
    zeros_rows = jnp.zeros((LAST_ROWS, D), jnp.float32)
    partials = _sc_edges(a, b, senders.astype(jnp.int32),
                         receivers.astype(jnp.int32), zeros_rows)
    return _combine(variables, partials, W_comb[:D, :], W_comb[D:, :],
                    b_comb.reshape(1, D))


# speedup vs baseline: 1.1857x; 1.0064x over previous
import functools

import jax
import jax.numpy as jnp
from jax import lax
from jax.experimental import pallas as pl
from jax.experimental.pallas import tpu as pltpu
from jax.experimental.pallas import tpu_sc as plsc

N_VARS = 10000
N_FACTORS = 10000
N_EDGES = 320000
D = 128

NC = 2
NS = 16
NW = NC * NS
EPW = N_EDGES // NW
C = 40
NCHUNK = EPW // C
SB = 25
NPLANE = 3
NSET = 4
SUB_ROWS = 624
LAST_ROWS = N_VARS - (NS - 1) * SUB_ROWS
LAST_OFF = (NS - 1) * SUB_ROWS

ROW_BLK = 2000
GRID = N_VARS // ROW_BLK


def _pre_body(v_ref, f_ref, w1_ref, w2_ref, bm_ref, a_ref, b_ref):
    a_ref[...] = jnp.dot(v_ref[...], w1_ref[...],
                         preferred_element_type=jnp.float32) + bm_ref[...]
    b_ref[...] = jnp.dot(f_ref[...], w2_ref[...],
                         preferred_element_type=jnp.float32)


def _pre(variables, factors, w1, w2, b_msg):
    return pl.pallas_call(
        _pre_body,
        grid=(GRID,),
        in_specs=[
            pl.BlockSpec((ROW_BLK, D), lambda i: (i, 0)),
            pl.BlockSpec((ROW_BLK, D), lambda i: (i, 0)),
            pl.BlockSpec((D, D), lambda i: (0, 0)),
            pl.BlockSpec((D, D), lambda i: (1, 0)),
            pl.BlockSpec((1, D), lambda i: (0, 0)),
        ],
        out_specs=[
            pl.BlockSpec((ROW_BLK, D), lambda i: (i, 0)),
            pl.BlockSpec((ROW_BLK, D), lambda i: (i, 0)),
        ],
        out_shape=[
            jax.ShapeDtypeStruct((N_VARS, D), jnp.float32),
            jax.ShapeDtypeStruct((N_FACTORS, D), jnp.float32),
        ],
    )(variables, factors, w1, w2, b_msg)


def _sc_body(a_hbm, b_hbm, snd_hbm, rcv_hbm, zeros_hbm, out_hbm,
             snd_v, rcv_v, a0, a1, a2, a3, b0, b1, b2, b3, acc,
             sg0, sg1, sg2, sg3, ss0, ss1, ss2, ss3):
    c = lax.axis_index("c")
    s = lax.axis_index("s")
    wid = c * NS + s

    pltpu.sync_copy(snd_hbm.at[wid, pl.ds(0, SB)], snd_v.at[0])
    pltpu.sync_copy(snd_hbm.at[wid, pl.ds(SB, SB)], snd_v.at[1])
    pltpu.sync_copy(rcv_hbm.at[wid, pl.ds(0, SB)], rcv_v.at[0])
    pltpu.sync_copy(rcv_hbm.at[wid, pl.ds(SB, SB)], rcv_v.at[1])

    abufs, bbufs = (a0, a1, a2, a3), (b0, b1, b2, b3)
    gsems, ssems = (sg0, sg1, sg2, sg3), (ss0, ss1, ss2, ss3)

    def sidx(x):
        return snd_v.at[(x // SB) % NPLANE, x % SB]

    def ridx(x):
        return rcv_v.at[(x // SB) % NPLANE, x % SB]

    HALves = ((0, 24), (24, 16))

    def shalf(x, off, ln):
        return snd_v.at[(x // SB) % NPLANE, x % SB, pl.ds(off, ln)]

    def rhalf(x, off, ln):
        return rcv_v.at[(x // SB) % NPLANE, x % SB, pl.ds(off, ln)]

    def issue_gathers(j, k):
        for off, ln in HALves:
            hs = pl.ds(off, ln)
            pltpu.async_copy(a_hbm.at[shalf(j, off, ln)], abufs[k].at[hs],
                             gsems[k])
            pltpu.async_copy(b_hbm.at[rhalf(j, off, ln)], bbufs[k].at[hs],
                             gsems[k])

    def wait_gathers(j, k):
        for off, ln in HALves:
            hs = pl.ds(off, ln)
            pltpu.make_async_copy(a_hbm.at[shalf(j, off, ln)],
                                  abufs[k].at[hs], gsems[k]).wait()
            pltpu.make_async_copy(b_hbm.at[rhalf(j, off, ln)],
                                  bbufs[k].at[hs], gsems[k]).wait()

    def compute(k):
        ab, bb = abufs[k], bbufs[k]

        def row(r, carry2):
            for l in range(D // 16):
                sl = pl.ds(l * 16, 16)
                bb[r, sl] = jnp.maximum(ab[r, sl] + bb[r, sl], 0.0)
            return carry2

        lax.fori_loop(0, C, row, 0)

    def issue_scatter(j, k):
        pltpu.async_copy(bbufs[k], acc.at[sidx(j)], ssems[k], add=True)

    def wait_scatter(j, k):
        pltpu.make_async_copy(bbufs[k], acc.at[sidx(j)], ssems[k]).wait()

    issue_gathers(0, 0)
    issue_gathers(1, 1)
    issue_gathers(2, 2)

    @pl.when(s < NS - 1)
    def _():
        pltpu.sync_copy(zeros_hbm.at[pl.ds(0, SUB_ROWS)],
                        acc.at[pl.ds(s * SUB_ROWS, SUB_ROWS)])

    @pl.when(s == NS - 1)
    def _():
        pltpu.sync_copy(zeros_hbm, acc.at[pl.ds(LAST_OFF, LAST_ROWS)])

    plsc.subcore_barrier()

    def quad(i, carry):
        for t in range(NSET):
            j = NSET * i + t

            @pl.when(j < NCHUNK)
            def _():
                wait_gathers(j, t)
                compute(t)
                issue_scatter(j, t)

                @pl.when(j >= 1)
                def _():
                    wait_scatter(j - 1, (t + 3) % NSET)

                @pl.when(jnp.logical_and((j + 3) % SB == 0,
                                         j + 3 < NCHUNK))
                def _():
                    x = j + 3
                    p = (x // SB) % NPLANE
                    pltpu.sync_copy(snd_hbm.at[wid, pl.ds(x, SB)],
                                    snd_v.at[p])
                    pltpu.sync_copy(rcv_hbm.at[wid, pl.ds(x, SB)],
                                    rcv_v.at[p])

                @pl.when(j + 3 < NCHUNK)
                def _():
                    issue_gathers(j + 3, (t + 3) % NSET)

        return carry

    lax.fori_loop(0, (NCHUNK + NSET - 1) // NSET, quad, 0)
    wait_scatter(NCHUNK - 1, (NCHUNK - 1) % NSET)
    plsc.subcore_barrier()

    @pl.when(s < NS - 1)
    def _():
        pltpu.sync_copy(acc.at[pl.ds(s * SUB_ROWS, SUB_ROWS)],
                        out_hbm.at[pl.ds(c * N_VARS + s * SUB_ROWS,
                                         SUB_ROWS)])

    @pl.when(s == NS - 1)
    def _():
        pltpu.sync_copy(acc.at[pl.ds(LAST_OFF, LAST_ROWS)],
                        out_hbm.at[pl.ds(c * N_VARS + LAST_OFF, LAST_ROWS)])


def _sc_edges(a, b, senders, receivers, zeros_rows):
    mesh = plsc.VectorSubcoreMesh(core_axis_name="c", subcore_axis_name="s")
    f = pl.kernel(
        _sc_body,
        out_type=jax.ShapeDtypeStruct((NC * N_VARS, D), jnp.float32),
        mesh=mesh,
        compiler_params=pltpu.CompilerParams(use_tc_tiling_on_sc=False),
        scratch_types=(
            [pltpu.VMEM((NPLANE, SB, C), jnp.int32)] * 2
            + [pltpu.VMEM((C, D), jnp.float32)] * (2 * NSET)
            + [pltpu.VMEM_SHARED((N_VARS, D), jnp.float32)]
            + [pltpu.SemaphoreType.DMA] * (2 * NSET)
        ),
    )
    return f(a, b, senders.reshape(NW, NCHUNK, C),
             receivers.reshape(NW, NCHUNK, C), zeros_rows)


def _comb_body(v_ref, p0_ref, p1_ref, wc1_ref, wc2_ref, bc_ref, o_ref):
    v = v_ref[...]
    aggr = p0_ref[...] + p1_ref[...]
    h = (jnp.dot(v, wc1_ref[...], preferred_element_type=jnp.float32)
         + jnp.dot(aggr, wc2_ref[...], preferred_element_type=jnp.float32)
         + bc_ref[...])
    o_ref[...] = v + jnp.maximum(h, 0.0)


def _combine(variables, partials, wc1, wc2, b_comb):
    return pl.pallas_call(
        _comb_body,
        grid=(GRID,),
        in_specs=[
            pl.BlockSpec((ROW_BLK, D), lambda i: (i, 0)),
            pl.BlockSpec((ROW_BLK, D), lambda i: (i, 0)),
            pl.BlockSpec((ROW_BLK, D), lambda i: (i + GRID, 0)),
            pl.BlockSpec((D, D), lambda i: (0, 0)),
            pl.BlockSpec((D, D), lambda i: (1, 0)),
            pl.BlockSpec((1, D), lambda i: (0, 0)),
        ],
        out_specs=pl.BlockSpec((ROW_BLK, D), lambda i: (i, 0)),
        out_shape=jax.ShapeDtypeStruct((N_VARS, D), jnp.float32),
    )(variables, partials, partials, wc1, wc2, b_comb)


def kernel(variables, factors, senders, receivers, edge_attr,
           W_msg, b_msg, W_comb, b_comb):
    del edge_attr
    a, b = _pre(variables, factors, W_msg, W_msg, b_msg.reshape(1, D))
    zeros_rows = jnp.zeros((LAST_ROWS, D), jnp.float32)
    partials = _sc_edges(a, b, senders.astype(jnp.int32),
                         receivers.astype(jnp.int32), zeros_rows)
    return _combine(variables, partials, W_comb, W_comb,
                    b_comb.reshape(1, D))
